# baseline (speedup 1.0000x reference)
import functools
import math

import jax
import jax.numpy as jnp
from jax import lax
from jax.experimental import pallas as pl
from jax.experimental.pallas import tpu as pltpu
from jax.experimental.pallas import tpu_sc as plsc

NC = 2
NS = 16
L = 16
NW = NC * NS

B, T, D = 128, 512, 768
N = B * T
SEG_PAD_ID = 2
TOK_PER_W = N // NW
C = 16
NSUB = TOK_PER_W // C
IDXW = 128
IDX_ROWS_W = TOK_PER_W // IDXW
NBUF = 4


def _build_base(pe2, w_seg):
    S = w_seg.shape[0]

    def body(pe_ref, seg_ref, out_ref):
        s = pl.program_id(0)
        out_ref[...] = pe_ref[...] + seg_ref[pl.ds(s, 1), :]

    return pl.pallas_call(
        body,
        grid=(S,),
        in_specs=[
            pl.BlockSpec((T, D), lambda s: (0, 0)),
            pl.BlockSpec((S, D), lambda s: (0, 0)),
        ],
        out_specs=pl.BlockSpec((T, D), lambda s: (s, 0)),
        out_shape=jax.ShapeDtypeStruct((S * T, D), jnp.float32),
    )(pe2, w_seg)


def _build_cidx(tt, am):
    def body(tt_ref, am_ref, out_ref):
        t = lax.broadcasted_iota(jnp.int32, (B, T), 1)
        ids = jnp.where(am_ref[...] == 0, SEG_PAD_ID, tt_ref[...])
        out_ref[...] = ids * T + t

    return pl.pallas_call(
        body,
        out_shape=jax.ShapeDtypeStruct((B, T), jnp.int32),
    )(tt, am)


def _sc_embed(idx2, cidx2, w_tok, base):
    scale = jnp.float32(math.sqrt(D))
    mesh = plsc.VectorSubcoreMesh(core_axis_name="c", subcore_axis_name="s")

    @functools.partial(
        pl.kernel,
        mesh=mesh,
        out_type=jax.ShapeDtypeStruct((N, D), jnp.float32),
        scratch_types=(
            [pltpu.VMEM((IDX_ROWS_W, IDXW), jnp.int32)] * 2
            + [pltpu.VMEM((C, D), jnp.float32)] * (2 * NBUF)
            + [pltpu.SemaphoreType.DMA] * (3 * NBUF)
        ),
    )
    def k(idx_hbm, cidx_hbm, wtok_hbm, base_hbm, out_hbm, *scr):
        idx_v, cidx_v = scr[0], scr[1]
        tok_bufs = scr[2:2 + NBUF]
        base_bufs = scr[2 + NBUF:2 + 2 * NBUF]
        sems = scr[2 + 2 * NBUF:]
        gt_sems = sems[0:NBUF]
        gb_sems = sems[NBUF:2 * NBUF]
        out_sems = sems[2 * NBUF:3 * NBUF]

        wid = lax.axis_index("s") * NC + lax.axis_index("c")
        row0 = wid * IDX_ROWS_W
        tok0 = wid * TOK_PER_W

        pltpu.sync_copy(idx_hbm.at[pl.ds(row0, IDX_ROWS_W)], idx_v)
        pltpu.sync_copy(cidx_hbm.at[pl.ds(row0, IDX_ROWS_W)], cidx_v)

        def _idx_slice(v, j):
            return v.at[j // (IDXW // C), pl.ds(lax.rem(j, IDXW // C) * C, C)]

        def issue_gathers(j, p):
            pltpu.async_copy(wtok_hbm.at[_idx_slice(idx_v, j)],
                             tok_bufs[p], gt_sems[p])
            pltpu.async_copy(base_hbm.at[_idx_slice(cidx_v, j)],
                             base_bufs[p], gb_sems[p])

        def wait_gathers(p):
            pltpu.make_async_copy(wtok_hbm.at[_idx_slice(idx_v, 0)],
                                  tok_bufs[p], gt_sems[p]).wait()
            pltpu.make_async_copy(base_hbm.at[_idx_slice(cidx_v, 0)],
                                  base_bufs[p], gb_sems[p]).wait()

        def wait_out(p):
            pltpu.make_async_copy(tok_bufs[p],
                                  out_hbm.at[pl.ds(tok0, C)], out_sems[p]).wait()

        def compute_and_out(j, p):
            def row_body(r, _):
                for cv in range(D // L):
                    sl = pl.ds(cv * L, L)
                    tok_bufs[p][r, sl] = (tok_bufs[p][r, sl] * scale
                                          + base_bufs[p][r, sl])
                return 0

            lax.fori_loop(0, C, row_body, 0)
            pltpu.async_copy(tok_bufs[p], out_hbm.at[pl.ds(tok0 + j * C, C)],
                             out_sems[p])

        issue_gathers(0, 0)
        issue_gathers(1, 1)

        def pipe_body(ii, _):
            for u in range(NBUF):
                i = NBUF * ii + u
                pf = (u + 2) % NBUF

                @pl.when(jnp.logical_and(i >= 2, i <= NSUB - 3))
                def _():
                    wait_out(pf)

                @pl.when(i <= NSUB - 3)
                def _():
                    issue_gathers(i + 2, pf)

                wait_gathers(u)
                compute_and_out(i, u)
            return 0

        lax.fori_loop(0, NSUB // NBUF, pipe_body, 0)
        for p in range(NBUF):
            wait_out(p)

    return k(idx2, cidx2, w_tok, base)


def kernel(inputs, token_type_ids, attn_mask, W_tok, W_seg, pe):
    pe2 = pe.reshape(T, D)
    base = _build_base(pe2, W_seg)
    cidx = _build_cidx(token_type_ids, attn_mask)
    idx2 = inputs.reshape(N // IDXW, IDXW)
    cidx2 = cidx.reshape(N // IDXW, IDXW)
    out = _sc_embed(idx2, cidx2, W_tok, base)
    return out.reshape(B, T, D)
